# baseline (speedup 1.0000x reference)
import functools
import math

import jax
import jax.numpy as jnp
from jax import lax
from jax.experimental import pallas as pl
from jax.experimental.pallas import tpu as pltpu
from jax.experimental.pallas import tpu_sc as plsc

VOCAB = 100000
EMBED = 32
BATCH = 1024


@functools.lru_cache(maxsize=None)
def _make_sc_gather(V4: int, B: int):
    info = plsc.get_sparse_core_info()
    NC, NS = info.num_cores, info.num_subcores
    NW = NC * NS
    b_per_w = B // NW
    L = info.num_lanes
    mesh = plsc.VectorSubcoreMesh(core_axis_name="c", subcore_axis_name="s")

    @functools.partial(
        pl.kernel,
        mesh=mesh,
        out_type=jax.ShapeDtypeStruct((B, 128), jnp.float32),
        scratch_types=[
            pltpu.VMEM((b_per_w,), jnp.int32),
            pltpu.VMEM((b_per_w,), jnp.int32),
            pltpu.VMEM((b_per_w, 128), jnp.float32),
            pltpu.SemaphoreType.DMA,
        ],
    )
    def gather(table_hbm, idx_hbm, out_hbm, idx_v, ridx_v, rows_v, sem):
        wid = lax.axis_index("s") * NC + lax.axis_index("c")
        base = wid * b_per_w
        pltpu.sync_copy(idx_hbm.at[pl.ds(base, b_per_w)], idx_v)
        for k in range(b_per_w // L):
            ridx_v[pl.ds(k * L, L)] = idx_v[pl.ds(k * L, L)] >> 2
        pltpu.async_copy(table_hbm.at[ridx_v], rows_v, sem).wait()
        pltpu.sync_copy(rows_v, out_hbm.at[pl.ds(base, b_per_w)])

    return gather


_TN = 2048


def _proj_body(x_ref, h4_ref, w_ref, b_ref, o_ref):
    sub = x_ref[...] & 3
    h4 = h4_ref[...]
    h = jnp.where(sub == 0, h4[:, 0:32], 0.0)
    h = h + jnp.where(sub == 1, h4[:, 32:64], 0.0)
    h = h + jnp.where(sub == 2, h4[:, 64:96], 0.0)
    h = h + jnp.where(sub == 3, h4[:, 96:128], 0.0)
    o_ref[...] = (
        jnp.dot(h, w_ref[...], preferred_element_type=jnp.float32) + b_ref[...]
    )


@functools.lru_cache(maxsize=None)
def _make_tc_proj(B: int, V: int):
    grid = (math.ceil(V / _TN),)
    return pl.pallas_call(
        _proj_body,
        grid=grid,
        in_specs=[
            pl.BlockSpec((B, 1), lambda j: (0, 0)),
            pl.BlockSpec((B, 128), lambda j: (0, 0)),
            pl.BlockSpec((EMBED, _TN), lambda j: (0, j)),
            pl.BlockSpec((1, _TN), lambda j: (0, j)),
        ],
        out_specs=pl.BlockSpec((B, _TN), lambda j: (0, j)),
        out_shape=jax.ShapeDtypeStruct((B, V), jnp.float32),
        compiler_params=pltpu.CompilerParams(
            dimension_semantics=("arbitrary",),
        ),
    )


def kernel(x, emb_table, W, b):
    x = x.astype(jnp.int32)
    table4 = emb_table.reshape(VOCAB // 4, 128)
    h4 = _make_sc_gather(VOCAB // 4, BATCH)(table4, x)
    proj = _make_tc_proj(BATCH, VOCAB)
    return proj(x.reshape(BATCH, 1), h4, W, b.reshape(1, VOCAB))


# speedup vs baseline: 1.0012x; 1.0012x over previous
import functools
import math

import jax
import jax.numpy as jnp
from jax import lax
from jax.experimental import pallas as pl
from jax.experimental.pallas import tpu as pltpu
from jax.experimental.pallas import tpu_sc as plsc

VOCAB = 100000
EMBED = 32
BATCH = 1024


@functools.lru_cache(maxsize=None)
def _make_sc_gather(V4: int, B: int):
    info = plsc.get_sparse_core_info()
    NC, NS = info.num_cores, info.num_subcores
    NW = NC * NS
    b_per_w = B // NW
    L = info.num_lanes
    mesh = plsc.VectorSubcoreMesh(core_axis_name="c", subcore_axis_name="s")

    @functools.partial(
        pl.kernel,
        mesh=mesh,
        out_type=jax.ShapeDtypeStruct((B, 128), jnp.float32),
        scratch_types=[
            pltpu.VMEM((b_per_w,), jnp.int32),
            pltpu.VMEM((b_per_w,), jnp.int32),
            pltpu.VMEM((b_per_w, 128), jnp.float32),
            pltpu.SemaphoreType.DMA,
        ],
    )
    def gather(table_hbm, idx_hbm, out_hbm, idx_v, ridx_v, rows_v, sem):
        wid = lax.axis_index("s") * NC + lax.axis_index("c")
        base = wid * b_per_w
        pltpu.sync_copy(idx_hbm.at[pl.ds(base, b_per_w)], idx_v)
        for k in range(b_per_w // L):
            ridx_v[pl.ds(k * L, L)] = idx_v[pl.ds(k * L, L)] >> 2
        pltpu.async_copy(table_hbm.at[ridx_v], rows_v, sem).wait()
        pltpu.sync_copy(rows_v, out_hbm.at[pl.ds(base, b_per_w)])

    return gather


_TN = 2048


def _proj_body(x_ref, h4_ref, w_ref, b_ref, o_ref):
    sub = x_ref[...] & 3
    h4 = h4_ref[...]
    h = jnp.where(sub == 0, h4[:, 0:32], 0.0)
    h = h + jnp.where(sub == 1, h4[:, 32:64], 0.0)
    h = h + jnp.where(sub == 2, h4[:, 64:96], 0.0)
    h = h + jnp.where(sub == 3, h4[:, 96:128], 0.0)
    o_ref[...] = (
        jnp.dot(h, w_ref[...], preferred_element_type=jnp.float32) + b_ref[...]
    )


@functools.lru_cache(maxsize=None)
def _make_tc_proj(B: int, V: int):
    grid = (math.ceil(V / _TN),)
    return pl.pallas_call(
        _proj_body,
        grid=grid,
        in_specs=[
            pl.BlockSpec((B, 1), lambda j: (0, 0)),
            pl.BlockSpec((B, 128), lambda j: (0, 0)),
            pl.BlockSpec((EMBED, _TN), lambda j: (0, j)),
            pl.BlockSpec((1, _TN), lambda j: (0, j)),
        ],
        out_specs=pl.BlockSpec((B, _TN), lambda j: (0, j)),
        out_shape=jax.ShapeDtypeStruct((B, V), jnp.float32),
        compiler_params=pltpu.CompilerParams(
            dimension_semantics=("parallel",),
        ),
    )


def kernel(x, emb_table, W, b):
    x = x.astype(jnp.int32)
    table4 = emb_table.reshape(VOCAB // 4, 128)
    h4 = _make_sc_gather(VOCAB // 4, BATCH)(table4, x)
    proj = _make_tc_proj(BATCH, VOCAB)
    return proj(x.reshape(BATCH, 1), h4, W, b.reshape(1, VOCAB))


# speedup vs baseline: 1.0065x; 1.0053x over previous
import functools
import math

import jax
import jax.numpy as jnp
from jax import lax
from jax.experimental import pallas as pl
from jax.experimental.pallas import tpu as pltpu
from jax.experimental.pallas import tpu_sc as plsc

VOCAB = 100000
EMBED = 32
BATCH = 1024


@functools.lru_cache(maxsize=None)
def _make_sc_gather(V4: int, B: int):
    info = plsc.get_sparse_core_info()
    NC, NS = info.num_cores, info.num_subcores
    NW = NC * NS
    b_per_w = B // NW
    L = info.num_lanes
    mesh = plsc.VectorSubcoreMesh(core_axis_name="c", subcore_axis_name="s")

    @functools.partial(
        pl.kernel,
        mesh=mesh,
        out_type=jax.ShapeDtypeStruct((B, 128), jnp.float32),
        scratch_types=[
            pltpu.VMEM((b_per_w,), jnp.int32),
            pltpu.VMEM((b_per_w,), jnp.int32),
            pltpu.VMEM((b_per_w, 128), jnp.float32),
            pltpu.SemaphoreType.DMA,
        ],
    )
    def gather(table_hbm, idx_hbm, out_hbm, idx_v, ridx_v, rows_v, sem):
        wid = lax.axis_index("s") * NC + lax.axis_index("c")
        base = wid * b_per_w
        pltpu.sync_copy(idx_hbm.at[pl.ds(base, b_per_w)], idx_v)
        for k in range(b_per_w // L):
            ridx_v[pl.ds(k * L, L)] = idx_v[pl.ds(k * L, L)] >> 2
        pltpu.async_copy(table_hbm.at[ridx_v], rows_v, sem).wait()
        pltpu.sync_copy(rows_v, out_hbm.at[pl.ds(base, b_per_w)])

    return gather


_BR = 32


def _proj_body(x_ref, h4_ref, w_ref, b_ref, o_ref):
    sub = x_ref[...] & 3
    h4 = h4_ref[...]
    h = jnp.where(sub == 0, h4[:, 0:32], 0.0)
    h = h + jnp.where(sub == 1, h4[:, 32:64], 0.0)
    h = h + jnp.where(sub == 2, h4[:, 64:96], 0.0)
    h = h + jnp.where(sub == 3, h4[:, 96:128], 0.0)
    o_ref[...] = (
        jnp.dot(h, w_ref[...], preferred_element_type=jnp.float32) + b_ref[...]
    )


@functools.lru_cache(maxsize=None)
def _make_tc_proj(B: int, V: int):
    grid = (B // _BR,)
    return pl.pallas_call(
        _proj_body,
        grid=grid,
        in_specs=[
            pl.BlockSpec((_BR, 1), lambda j: (j, 0)),
            pl.BlockSpec((_BR, 128), lambda j: (j, 0)),
            pl.BlockSpec((EMBED, V), lambda j: (0, 0)),
            pl.BlockSpec((1, V), lambda j: (0, 0)),
        ],
        out_specs=pl.BlockSpec((_BR, V), lambda j: (j, 0)),
        out_shape=jax.ShapeDtypeStruct((B, V), jnp.float32),
        compiler_params=pltpu.CompilerParams(
            dimension_semantics=("parallel",),
        ),
    )


def kernel(x, emb_table, W, b):
    x = x.astype(jnp.int32)
    table4 = emb_table.reshape(VOCAB // 4, 128)
    h4 = _make_sc_gather(VOCAB // 4, BATCH)(table4, x)
    proj = _make_tc_proj(BATCH, VOCAB)
    return proj(x.reshape(BATCH, 1), h4, W, b.reshape(1, VOCAB))


# speedup vs baseline: 3.1397x; 3.1195x over previous
import functools
import math

import jax
import jax.numpy as jnp
from jax import lax
from jax.experimental import pallas as pl
from jax.experimental.pallas import tpu as pltpu
from jax.experimental.pallas import tpu_sc as plsc

VOCAB = 100000
EMBED = 32
BATCH = 1024
_CH = 128
_RND = 16


@functools.lru_cache(maxsize=None)
def _make_sc_gather(V: int, E: int, B: int):
    info = plsc.get_sparse_core_info()
    NC, NS, L = info.num_cores, info.num_subcores, info.num_lanes
    NW = NC * NS
    b_per_w = B // NW
    mesh = plsc.VectorSubcoreMesh(core_axis_name="c", subcore_axis_name="s")

    @functools.partial(
        pl.kernel,
        mesh=mesh,
        out_type=jax.ShapeDtypeStruct((B, E, _CH), jnp.float32),
        scratch_types=[
            pltpu.VMEM((b_per_w,), jnp.int32),
            pltpu.VMEM_SHARED((NS, b_per_w), jnp.int32),
            pltpu.SMEM((b_per_w,), jnp.int32),
            pltpu.VMEM((_RND * E, _CH), jnp.float32),
            pltpu.SemaphoreType.DMA,
        ],
    )
    def gather(tbl_hbm, idx_hbm, out_hbm, idx_v, idx_sp, idx_s, buf_v, sem):
        sid = lax.axis_index("s")
        wid = sid * NC + lax.axis_index("c")
        base = wid * b_per_w
        pltpu.sync_copy(idx_hbm.at[pl.ds(base, b_per_w)], idx_v)
        pltpu.sync_copy(idx_v, idx_sp.at[sid])
        pltpu.sync_copy(idx_sp.at[sid], idx_s)
        for r in range(b_per_w // _RND):
            descs = []
            for j in range(_RND):
                k = r * _RND + j
                a = pl.multiple_of((idx_s[k] >> 7) << 7, _CH)
                descs.append(
                    pltpu.async_copy(
                        tbl_hbm.at[:, pl.ds(a, _CH)],
                        buf_v.at[pl.ds(j * E, E)],
                        sem,
                    )
                )
            for d in descs:
                d.wait()
            descs = []
            for j in range(_RND):
                k = r * _RND + j
                descs.append(
                    pltpu.async_copy(
                        buf_v.at[pl.ds(j * E, E)], out_hbm.at[base + k], sem
                    )
                )
            for d in descs:
                d.wait()

    return gather


def _extract_body(x_ref, h128_ref, h_ref):
    c = x_ref[...] & (_CH - 1)
    lanes = lax.broadcasted_iota(jnp.int32, (1, 1, _CH), 2)
    m = c[:, :, None] == lanes
    h_ref[...] = jnp.sum(jnp.where(m, h128_ref[...], 0.0), axis=2)


@functools.lru_cache(maxsize=None)
def _make_tc_extract(B: int, E: int):
    return pl.pallas_call(
        _extract_body,
        out_shape=jax.ShapeDtypeStruct((B, E), jnp.float32),
    )


_TN = 2048


def _proj_body(h_ref, w_ref, b_ref, o_ref):
    prod = lax.dot_general(
        w_ref[...], h_ref[...],
        (((0,), (1,)), ((), ())),
        preferred_element_type=jnp.float32,
    )
    ones = jnp.ones((1, o_ref.shape[1]), jnp.float32)
    bias = lax.dot_general(
        b_ref[...], ones,
        (((0,), (0,)), ((), ())),
        preferred_element_type=jnp.float32,
    )
    o_ref[...] = prod + bias


@functools.lru_cache(maxsize=None)
def _make_tc_proj(B: int, V: int):
    grid = (math.ceil(V / _TN),)
    return pl.pallas_call(
        _proj_body,
        grid=grid,
        in_specs=[
            pl.BlockSpec((B, EMBED), lambda j: (0, 0)),
            pl.BlockSpec((EMBED, _TN), lambda j: (0, j)),
            pl.BlockSpec((1, _TN), lambda j: (0, j)),
        ],
        out_specs=pl.BlockSpec((_TN, B), lambda j: (j, 0)),
        out_shape=jax.ShapeDtypeStruct((V, B), jnp.float32),
        compiler_params=pltpu.CompilerParams(
            dimension_semantics=("parallel",),
        ),
    )


def kernel(x, emb_table, W, b):
    x = x.astype(jnp.int32)
    tbl_t = emb_table.T
    h128 = _make_sc_gather(VOCAB, EMBED, BATCH)(tbl_t, x)
    h = _make_tc_extract(BATCH, EMBED)(x.reshape(BATCH, 1), h128)
    out_t = _make_tc_proj(BATCH, VOCAB)(h, W, b.reshape(1, VOCAB))
    return out_t.T


# speedup vs baseline: 3.1683x; 1.0091x over previous
import functools
import math

import jax
import jax.numpy as jnp
from jax import lax
from jax.experimental import pallas as pl
from jax.experimental.pallas import tpu as pltpu
from jax.experimental.pallas import tpu_sc as plsc

VOCAB = 100000
EMBED = 32
BATCH = 1024
_CH = 128
_RND = 16


@functools.lru_cache(maxsize=None)
def _make_sc_gather(V: int, E: int, B: int):
    info = plsc.get_sparse_core_info()
    NC, NS, L = info.num_cores, info.num_subcores, info.num_lanes
    NW = NC * NS
    b_per_w = B // NW
    mesh = plsc.VectorSubcoreMesh(core_axis_name="c", subcore_axis_name="s")

    @functools.partial(
        pl.kernel,
        mesh=mesh,
        out_type=jax.ShapeDtypeStruct((B, E, _CH), jnp.float32),
        scratch_types=[
            pltpu.VMEM((b_per_w,), jnp.int32),
            pltpu.VMEM_SHARED((NS, b_per_w), jnp.int32),
            pltpu.SMEM((b_per_w,), jnp.int32),
            pltpu.VMEM((_RND * E, _CH), jnp.float32),
            pltpu.SemaphoreType.DMA,
        ],
    )
    def gather(tbl_hbm, idx_hbm, out_hbm, idx_v, idx_sp, idx_s, buf_v, sem):
        sid = lax.axis_index("s")
        wid = sid * NC + lax.axis_index("c")
        base = wid * b_per_w
        pltpu.sync_copy(idx_hbm.at[pl.ds(base, b_per_w)], idx_v)
        pltpu.sync_copy(idx_v, idx_sp.at[sid])
        pltpu.sync_copy(idx_sp.at[sid], idx_s)
        for r in range(b_per_w // _RND):
            descs = []
            for j in range(_RND):
                k = r * _RND + j
                a = pl.multiple_of((idx_s[k] >> 7) << 7, _CH)
                descs.append(
                    pltpu.async_copy(
                        tbl_hbm.at[:, pl.ds(a, _CH)],
                        buf_v.at[pl.ds(j * E, E)],
                        sem,
                    )
                )
            for d in descs:
                d.wait()
            descs = []
            for j in range(_RND):
                k = r * _RND + j
                descs.append(
                    pltpu.async_copy(
                        buf_v.at[pl.ds(j * E, E)], out_hbm.at[base + k], sem
                    )
                )
            for d in descs:
                d.wait()

    return gather


_XB = 128


def _extract_body(x_ref, h128_ref, h_ref):
    c = x_ref[...] & (_CH - 1)
    lanes = lax.broadcasted_iota(jnp.int32, (1, 1, _CH), 2)
    m = c[:, :, None] == lanes
    h_ref[...] = jnp.sum(jnp.where(m, h128_ref[...], 0.0), axis=2)


@functools.lru_cache(maxsize=None)
def _make_tc_extract(B: int, E: int):
    return pl.pallas_call(
        _extract_body,
        grid=(B // _XB,),
        in_specs=[
            pl.BlockSpec((_XB, 1), lambda j: (j, 0)),
            pl.BlockSpec((_XB, E, _CH), lambda j: (j, 0, 0)),
        ],
        out_specs=pl.BlockSpec((_XB, E), lambda j: (j, 0)),
        out_shape=jax.ShapeDtypeStruct((B, E), jnp.float32),
        compiler_params=pltpu.CompilerParams(
            dimension_semantics=("parallel",),
        ),
    )


_TN = 2048


def _proj_body(h_ref, w_ref, b_ref, o_ref):
    prod = lax.dot_general(
        w_ref[...], h_ref[...],
        (((0,), (1,)), ((), ())),
        preferred_element_type=jnp.float32,
    )
    ones = jnp.ones((1, o_ref.shape[1]), jnp.float32)
    bias = lax.dot_general(
        b_ref[...], ones,
        (((0,), (0,)), ((), ())),
        preferred_element_type=jnp.float32,
    )
    o_ref[...] = prod + bias


@functools.lru_cache(maxsize=None)
def _make_tc_proj(B: int, V: int):
    grid = (math.ceil(V / _TN),)
    return pl.pallas_call(
        _proj_body,
        grid=grid,
        in_specs=[
            pl.BlockSpec((B, EMBED), lambda j: (0, 0)),
            pl.BlockSpec((EMBED, _TN), lambda j: (0, j)),
            pl.BlockSpec((1, _TN), lambda j: (0, j)),
        ],
        out_specs=pl.BlockSpec((_TN, B), lambda j: (j, 0)),
        out_shape=jax.ShapeDtypeStruct((V, B), jnp.float32),
        compiler_params=pltpu.CompilerParams(
            dimension_semantics=("parallel",),
        ),
    )


def kernel(x, emb_table, W, b):
    x = x.astype(jnp.int32)
    tbl_t = emb_table.T
    h128 = _make_sc_gather(VOCAB, EMBED, BATCH)(tbl_t, x)
    h = _make_tc_extract(BATCH, EMBED)(x.reshape(BATCH, 1), h128)
    out_t = _make_tc_proj(BATCH, VOCAB)(h, W, b.reshape(1, VOCAB))
    return out_t.T
